# Initial kernel scaffold; baseline (speedup 1.0000x reference)
import jax
import jax.numpy as jnp
from jax.experimental import pallas as pl


def kernel(x, edge_index, W0, b0, W1, b1, Wout, bout):
    raise NotImplementedError("write your pallas kernel here")


# speedup vs baseline: 5.9786x; 5.9786x over previous
import functools

import jax
import jax.numpy as jnp
from jax import lax
from jax.experimental import pallas as pl
from jax.experimental.pallas import tpu as pltpu
from jax.experimental.pallas import tpu_sc as plsc

N_NODES = 10000
N_EDGES = 320000
D = 128
WOUT_PAD = 64
N_CLASS = 41

NC = 2
NS = 16
NW = NC * NS

NP = 10240
ROWS_PT = NP // NS
EPT = N_EDGES // NW
CHUNK = 128
NFULL = EPT // CHUNK
TAIL = EPT - NFULL * CHUNK


def _make_edge_scatter(W):
  mesh = plsc.VectorSubcoreMesh(core_axis_name="c", subcore_axis_name="s")

  @functools.partial(
      pl.kernel,
      mesh=mesh,
      out_type=jax.ShapeDtypeStruct((NC * NP, W), jnp.float32),
      scratch_types=[
          pltpu.VMEM((CHUNK,), jnp.int32),
          pltpu.VMEM((CHUNK,), jnp.int32),
          pltpu.VMEM((TAIL,), jnp.int32),
          pltpu.VMEM((TAIL,), jnp.int32),
          pltpu.VMEM((CHUNK, W), jnp.float32),
          pltpu.VMEM((TAIL, W), jnp.float32),
          pltpu.VMEM((16, W), jnp.float32),
          pltpu.VMEM_SHARED((NP, W), jnp.float32),
          pltpu.SemaphoreType.DMA,
      ],
  )
  def scatter_kernel(support, src, dst, out,
                     src_v, dst_v, tsrc_v, tdst_v, rows_v, trows_v,
                     zero_v, agg, sem):
    cid = lax.axis_index("c")
    sid = lax.axis_index("s")

    zero16 = jnp.zeros((16,), jnp.float32)
    for r in range(16):
      for q in range(W // 16):
        zero_v[r, pl.ds(q * 16, 16)] = zero16
    rbase = sid * ROWS_PT

    def zero_body(i, _):
      pltpu.sync_copy(zero_v, agg.at[pl.ds(rbase + i * 16, 16)])
      return _
    lax.fori_loop(0, ROWS_PT // 16, zero_body, None)
    plsc.subcore_barrier()

    ebase = (cid * NS + sid) * EPT

    def edge_body(j, _):
      e0 = ebase + j * CHUNK
      pltpu.sync_copy(src.at[pl.ds(e0, CHUNK)], src_v)
      pltpu.sync_copy(dst.at[pl.ds(e0, CHUNK)], dst_v)
      pltpu.async_copy(support.at[src_v], rows_v, sem).wait()
      pltpu.sync_copy(rows_v, agg.at[dst_v], add=True)
      return _
    lax.fori_loop(0, NFULL, edge_body, None)

    if TAIL:
      e0 = ebase + NFULL * CHUNK
      pltpu.sync_copy(src.at[pl.ds(e0, TAIL)], tsrc_v)
      pltpu.sync_copy(dst.at[pl.ds(e0, TAIL)], tdst_v)
      pltpu.async_copy(support.at[tsrc_v], trows_v, sem).wait()
      pltpu.sync_copy(trows_v, agg.at[tdst_v], add=True)
    plsc.subcore_barrier()

    obase = cid * NP + rbase

    def wb_body(i, _):
      pltpu.sync_copy(agg.at[pl.ds(rbase + i * CHUNK, CHUNK)], rows_v)
      pltpu.sync_copy(rows_v, out.at[pl.ds(obase + i * CHUNK, CHUNK)])
      return _
    lax.fori_loop(0, ROWS_PT // CHUNK, wb_body, None)

  return scatter_kernel


_scatter128 = _make_edge_scatter(D)

_BR = 1024


def _mm_body(x_ref, w_ref, o_ref):
  o_ref[...] = jnp.dot(x_ref[...], w_ref[...],
                       preferred_element_type=jnp.float32)


def _combine_mm_body(p0_ref, p1_ref, b_ref, w_ref, o_ref):
  h = jnp.maximum(p0_ref[...] + p1_ref[...] + b_ref[...], 0.0)
  o_ref[...] = jnp.dot(h, w_ref[...], preferred_element_type=jnp.float32)


def _combine_body(p0_ref, p1_ref, b_ref, o_ref):
  o_ref[...] = jnp.maximum(p0_ref[...] + p1_ref[...] + b_ref[...], 0.0)


def _final_body(p0_ref, p1_ref, w_ref, b_ref, o_ref):
  agg = p0_ref[...] + p1_ref[...]
  v = jnp.dot(agg, w_ref[...],
              preferred_element_type=jnp.float32) + b_ref[...]
  col = lax.broadcasted_iota(jnp.int32, v.shape, 1)
  valid = col < N_CLASS
  vm = jnp.where(valid, v, -jnp.inf)
  m = jnp.max(vm, axis=1, keepdims=True)
  ex = jnp.where(valid, jnp.exp(v - m), 0.0)
  lse = jnp.log(jnp.sum(ex, axis=1, keepdims=True)) + m
  o_ref[...] = v - lse


def _tc_matmul(x, w):
  n, k = x.shape
  kw, m = w.shape
  return pl.pallas_call(
      _mm_body,
      grid=(n // _BR,),
      in_specs=[pl.BlockSpec((_BR, k), lambda i: (i, 0)),
                pl.BlockSpec((kw, m), lambda i: (0, 0))],
      out_specs=pl.BlockSpec((_BR, m), lambda i: (i, 0)),
      out_shape=jax.ShapeDtypeStruct((n, m), jnp.float32),
  )(x, w)


def _tc_combine_mm(p0, p1, b, w):
  n, k = p0.shape
  kw, m = w.shape
  return pl.pallas_call(
      _combine_mm_body,
      grid=(n // _BR,),
      in_specs=[pl.BlockSpec((_BR, k), lambda i: (i, 0)),
                pl.BlockSpec((_BR, k), lambda i: (i, 0)),
                pl.BlockSpec((1, k), lambda i: (0, 0)),
                pl.BlockSpec((kw, m), lambda i: (0, 0))],
      out_specs=pl.BlockSpec((_BR, m), lambda i: (i, 0)),
      out_shape=jax.ShapeDtypeStruct((n, m), jnp.float32),
  )(p0, p1, b, w)


def _tc_combine(p0, p1, b):
  n, k = p0.shape
  return pl.pallas_call(
      _combine_body,
      grid=(n // _BR,),
      in_specs=[pl.BlockSpec((_BR, k), lambda i: (i, 0)),
                pl.BlockSpec((_BR, k), lambda i: (i, 0)),
                pl.BlockSpec((1, k), lambda i: (0, 0))],
      out_specs=pl.BlockSpec((_BR, k), lambda i: (i, 0)),
      out_shape=jax.ShapeDtypeStruct((n, k), jnp.float32),
  )(p0, p1, b)


def _tc_final(p0, p1, w, b):
  n, k = p0.shape
  kw, m = w.shape
  return pl.pallas_call(
      _final_body,
      grid=(n // _BR,),
      in_specs=[pl.BlockSpec((_BR, k), lambda i: (i, 0)),
                pl.BlockSpec((_BR, k), lambda i: (i, 0)),
                pl.BlockSpec((kw, m), lambda i: (0, 0)),
                pl.BlockSpec((1, m), lambda i: (0, 0))],
      out_specs=pl.BlockSpec((_BR, m), lambda i: (i, 0)),
      out_shape=jax.ShapeDtypeStruct((n, m), jnp.float32),
  )(p0, p1, w, b)


def kernel(x, edge_index, W0, b0, W1, b1, Wout, bout):
  src = edge_index[0].astype(jnp.int32)
  dst = edge_index[1].astype(jnp.int32)
  xp = jnp.pad(x, ((0, NP - N_NODES), (0, 0)))
  wout_p = jnp.pad(Wout, ((0, 0), (0, WOUT_PAD - N_CLASS)))
  bout_p = jnp.pad(bout, (0, WOUT_PAD - N_CLASS)).reshape(1, WOUT_PAD)
  b0r = b0.reshape(1, D)
  b1r = b1.reshape(1, D)

  support0 = _tc_matmul(xp, W0)
  p0 = _scatter128(support0, src, dst)
  support1 = _tc_combine_mm(p0[:NP], p0[NP:], b0r, W1)
  p1 = _scatter128(support1, src, dst)
  h2 = _tc_combine(p1[:NP], p1[NP:], b1r)
  p2 = _scatter128(h2, src, dst)
  pred = _tc_final(p2[:NP], p2[NP:], wout_p, bout_p)
  return pred[:N_NODES, :N_CLASS]


# speedup vs baseline: 12.5648x; 2.1016x over previous
import functools

import jax
import jax.numpy as jnp
from jax import lax
from jax.experimental import pallas as pl
from jax.experimental.pallas import tpu as pltpu
from jax.experimental.pallas import tpu_sc as plsc

N_NODES = 10000
N_EDGES = 320000
D = 128
WOUT_PAD = 64
N_CLASS = 41

NC = 2
NS = 16
NW = NC * NS

NP = 10240
ROWS_PT = NP // NS
EPT = N_EDGES // NW
CHUNK = 80
NCHUNK = EPT // CHUNK
NBUF = 4
GDEPTH = 2


def _make_edge_scatter(W):
  mesh = plsc.VectorSubcoreMesh(core_axis_name="c", subcore_axis_name="s")

  rows_t = [pltpu.VMEM((CHUNK, W), jnp.float32) for _ in range(NBUF)]
  sidx_t = [pltpu.VMEM((CHUNK,), jnp.int32) for _ in range(NBUF)]
  didx_t = [pltpu.VMEM((CHUNK,), jnp.int32) for _ in range(NBUF)]
  sem_t = [pltpu.SemaphoreType.DMA for _ in range(3 * NBUF)]

  @functools.partial(
      pl.kernel,
      mesh=mesh,
      out_type=jax.ShapeDtypeStruct((NC * NP, W), jnp.float32),
      scratch_types=[
          pltpu.VMEM((16, W), jnp.float32),
          pltpu.VMEM_SHARED((NP, W), jnp.float32),
      ] + rows_t + sidx_t + didx_t + sem_t,
  )
  def scatter_kernel(support, src3d, dst3d, out, zero_v, agg, *bufs):
    rows = bufs[:NBUF]
    sidx = bufs[NBUF:2 * NBUF]
    didx = bufs[2 * NBUF:3 * NBUF]
    gsem = bufs[3 * NBUF:4 * NBUF]
    ssem = bufs[4 * NBUF:5 * NBUF]
    isem = bufs[5 * NBUF:6 * NBUF]
    cid = lax.axis_index("c")
    sid = lax.axis_index("s")
    wid = cid * NS + sid

    zero16 = jnp.zeros((16,), jnp.float32)
    for r in range(16):
      for q in range(W // 16):
        zero_v[r, pl.ds(q * 16, 16)] = zero16
    rbase = sid * ROWS_PT

    def zero_body(i, _):
      pltpu.sync_copy(zero_v, agg.at[pl.ds(rbase + i * 16, 16)])
      return _
    lax.fori_loop(0, ROWS_PT // 16, zero_body, None)
    plsc.subcore_barrier()

    ebase = wid * EPT

    def i_start(j, b):
      e0 = ebase + j * CHUNK
      pltpu.async_copy(src3d.at[pl.ds(e0, CHUNK)], sidx[b], isem[b])
      pltpu.async_copy(dst3d.at[pl.ds(e0, CHUNK)], didx[b], isem[b])

    def i_wait(j, b):
      e0 = ebase + j * CHUNK
      pltpu.make_async_copy(src3d.at[pl.ds(e0, CHUNK)], sidx[b],
                            isem[b]).wait()
      pltpu.make_async_copy(dst3d.at[pl.ds(e0, CHUNK)], didx[b],
                            isem[b]).wait()

    def g_start(j, b):
      pltpu.async_copy(support.at[sidx[b]], rows[b], gsem[b])

    def g_wait(j, b):
      pltpu.make_async_copy(support.at[sidx[b]], rows[b], gsem[b]).wait()

    def s_start(j, b):
      pltpu.async_copy(rows[b], agg.at[didx[b]], ssem[b], add=True)

    def s_wait(j, b):
      pltpu.make_async_copy(rows[b], agg.at[didx[b]], ssem[b]).wait()

    def slot(j, b, swait, gstart, istart):
      g_wait(j, b)
      s_start(j, b)
      if swait:
        s_wait(j - GDEPTH, (b + GDEPTH) % NBUF)
      if gstart:
        i_wait(j + GDEPTH, (b + GDEPTH) % NBUF)
        g_start(j + GDEPTH, (b + GDEPTH) % NBUF)
      if istart:
        i_start(j + GDEPTH + 1, (b + GDEPTH + 1) % NBUF)

    for j in range(GDEPTH + 1):
      i_start(j, j % NBUF)
    for j in range(GDEPTH):
      i_wait(j, j % NBUF)
      g_start(j, j % NBUF)

    for j in range(NBUF):
      slot(j, j % NBUF, swait=(j >= GDEPTH), gstart=True, istart=True)

    def ring_body(g, _):
      j0 = g * NBUF
      for b in range(NBUF):
        slot(j0 + b, b, swait=True, gstart=True, istart=True)
      return _
    lax.fori_loop(1, (NCHUNK - NBUF - 1) // NBUF, ring_body, None)

    for j in range(((NCHUNK - NBUF - 1) // NBUF) * NBUF, NCHUNK):
      slot(j, j % NBUF, swait=True,
           gstart=(j + GDEPTH < NCHUNK),
           istart=(j + GDEPTH + 1 < NCHUNK))
    for j in range(NCHUNK - GDEPTH, NCHUNK):
      s_wait(j, j % NBUF)
    plsc.subcore_barrier()

    obase = cid * NP + rbase

    def wb_body(i, _):
      pltpu.sync_copy(agg.at[pl.ds(rbase + i * CHUNK, CHUNK)], rows[0])
      pltpu.sync_copy(rows[0], out.at[pl.ds(obase + i * CHUNK, CHUNK)])
      return _
    lax.fori_loop(0, ROWS_PT // CHUNK, wb_body, None)

  return scatter_kernel


_scatter128 = _make_edge_scatter(D)

_BR = 1024


def _mm_body(x_ref, w_ref, o_ref):
  o_ref[...] = jnp.dot(x_ref[...], w_ref[...],
                       preferred_element_type=jnp.float32)


def _combine_mm_body(p0_ref, p1_ref, b_ref, w_ref, o_ref):
  h = jnp.maximum(p0_ref[...] + p1_ref[...] + b_ref[...], 0.0)
  o_ref[...] = jnp.dot(h, w_ref[...], preferred_element_type=jnp.float32)


def _combine_body(p0_ref, p1_ref, b_ref, o_ref):
  o_ref[...] = jnp.maximum(p0_ref[...] + p1_ref[...] + b_ref[...], 0.0)


def _final_body(p0_ref, p1_ref, w_ref, b_ref, o_ref):
  agg = p0_ref[...] + p1_ref[...]
  v = jnp.dot(agg, w_ref[...],
              preferred_element_type=jnp.float32) + b_ref[...]
  col = lax.broadcasted_iota(jnp.int32, v.shape, 1)
  valid = col < N_CLASS
  vm = jnp.where(valid, v, -jnp.inf)
  m = jnp.max(vm, axis=1, keepdims=True)
  ex = jnp.where(valid, jnp.exp(v - m), 0.0)
  lse = jnp.log(jnp.sum(ex, axis=1, keepdims=True)) + m
  o_ref[...] = v - lse


def _tc_matmul(x, w):
  n, k = x.shape
  kw, m = w.shape
  return pl.pallas_call(
      _mm_body,
      grid=(n // _BR,),
      in_specs=[pl.BlockSpec((_BR, k), lambda i: (i, 0)),
                pl.BlockSpec((kw, m), lambda i: (0, 0))],
      out_specs=pl.BlockSpec((_BR, m), lambda i: (i, 0)),
      out_shape=jax.ShapeDtypeStruct((n, m), jnp.float32),
  )(x, w)


def _tc_combine_mm(p0, p1, b, w):
  n, k = p0.shape
  kw, m = w.shape
  return pl.pallas_call(
      _combine_mm_body,
      grid=(n // _BR,),
      in_specs=[pl.BlockSpec((_BR, k), lambda i: (i, 0)),
                pl.BlockSpec((_BR, k), lambda i: (i, 0)),
                pl.BlockSpec((1, k), lambda i: (0, 0)),
                pl.BlockSpec((kw, m), lambda i: (0, 0))],
      out_specs=pl.BlockSpec((_BR, m), lambda i: (i, 0)),
      out_shape=jax.ShapeDtypeStruct((n, m), jnp.float32),
  )(p0, p1, b, w)


def _tc_combine(p0, p1, b):
  n, k = p0.shape
  return pl.pallas_call(
      _combine_body,
      grid=(n // _BR,),
      in_specs=[pl.BlockSpec((_BR, k), lambda i: (i, 0)),
                pl.BlockSpec((_BR, k), lambda i: (i, 0)),
                pl.BlockSpec((1, k), lambda i: (0, 0))],
      out_specs=pl.BlockSpec((_BR, k), lambda i: (i, 0)),
      out_shape=jax.ShapeDtypeStruct((n, k), jnp.float32),
  )(p0, p1, b)


def _tc_final(p0, p1, w, b):
  n, k = p0.shape
  kw, m = w.shape
  return pl.pallas_call(
      _final_body,
      grid=(n // _BR,),
      in_specs=[pl.BlockSpec((_BR, k), lambda i: (i, 0)),
                pl.BlockSpec((_BR, k), lambda i: (i, 0)),
                pl.BlockSpec((kw, m), lambda i: (0, 0)),
                pl.BlockSpec((1, m), lambda i: (0, 0))],
      out_specs=pl.BlockSpec((_BR, m), lambda i: (i, 0)),
      out_shape=jax.ShapeDtypeStruct((n, m), jnp.float32),
  )(p0, p1, w, b)


def kernel(x, edge_index, W0, b0, W1, b1, Wout, bout):
  src = edge_index[0].astype(jnp.int32)
  dst = edge_index[1].astype(jnp.int32)
  xp = jnp.pad(x, ((0, NP - N_NODES), (0, 0)))
  wout_p = jnp.pad(Wout, ((0, 0), (0, WOUT_PAD - N_CLASS)))
  bout_p = jnp.pad(bout, (0, WOUT_PAD - N_CLASS)).reshape(1, WOUT_PAD)
  b0r = b0.reshape(1, D)
  b1r = b1.reshape(1, D)

  support0 = _tc_matmul(xp, W0)
  p0 = _scatter128(support0, src, dst)
  support1 = _tc_combine_mm(p0[:NP], p0[NP:], b0r, W1)
  p1 = _scatter128(support1, src, dst)
  h2 = _tc_combine(p1[:NP], p1[NP:], b1r)
  p2 = _scatter128(h2, src, dst)
  pred = _tc_final(p2[:NP], p2[NP:], wout_p, bout_p)
  return pred[:N_NODES, :N_CLASS]


# speedup vs baseline: 12.8726x; 1.0245x over previous
import functools

import jax
import jax.numpy as jnp
from jax import lax
from jax.experimental import pallas as pl
from jax.experimental.pallas import tpu as pltpu
from jax.experimental.pallas import tpu_sc as plsc

N_NODES = 10000
N_EDGES = 320000
D = 128
WOUT_PAD = 64
N_CLASS = 41

NC = 2
NS = 16
NW = NC * NS

NP = 10240
ROWS_PT = NP // NS
EPT = N_EDGES // NW
CHUNK = 80
NCHUNK = EPT // CHUNK
NBUF = 4
GDEPTH = 2


def _make_edge_scatter(W):
  mesh = plsc.VectorSubcoreMesh(core_axis_name="c", subcore_axis_name="s")

  rows_t = [pltpu.VMEM((CHUNK, W), jnp.float32) for _ in range(NBUF)]
  sidx_t = [pltpu.VMEM((CHUNK,), jnp.int32) for _ in range(NBUF)]
  didx_t = [pltpu.VMEM((CHUNK,), jnp.int32) for _ in range(NBUF)]
  sem_t = [pltpu.SemaphoreType.DMA for _ in range(3 * NBUF)]

  @functools.partial(
      pl.kernel,
      mesh=mesh,
      out_type=jax.ShapeDtypeStruct((NC * NP, W), jnp.float32),
      scratch_types=[
          pltpu.VMEM_SHARED((NP, W), jnp.float32),
      ] + rows_t + sidx_t + didx_t + sem_t,
  )
  def scatter_kernel(support, src3d, dst3d, out, agg, *bufs):
    rows = bufs[:NBUF]
    sidx = bufs[NBUF:2 * NBUF]
    didx = bufs[2 * NBUF:3 * NBUF]
    gsem = bufs[3 * NBUF:4 * NBUF]
    ssem = bufs[4 * NBUF:5 * NBUF]
    isem = bufs[5 * NBUF:6 * NBUF]
    cid = lax.axis_index("c")
    sid = lax.axis_index("s")
    wid = cid * NS + sid
    rbase = sid * ROWS_PT

    ebase = wid * EPT

    def i_start(j, b):
      e0 = ebase + j * CHUNK
      pltpu.async_copy(src3d.at[pl.ds(e0, CHUNK)], sidx[b], isem[b])
      pltpu.async_copy(dst3d.at[pl.ds(e0, CHUNK)], didx[b], isem[b])

    def i_wait(j, b):
      e0 = ebase + j * CHUNK
      pltpu.make_async_copy(src3d.at[pl.ds(e0, CHUNK)], sidx[b],
                            isem[b]).wait()
      pltpu.make_async_copy(dst3d.at[pl.ds(e0, CHUNK)], didx[b],
                            isem[b]).wait()

    def g_start(j, b):
      pltpu.async_copy(support.at[sidx[b]], rows[b], gsem[b])

    def g_wait(j, b):
      pltpu.make_async_copy(support.at[sidx[b]], rows[b], gsem[b]).wait()

    def s_start(j, b):
      pltpu.async_copy(rows[b], agg.at[didx[b]], ssem[b], add=True)

    def s_wait(j, b):
      pltpu.make_async_copy(rows[b], agg.at[didx[b]], ssem[b]).wait()

    def slot(j, b, swait, gstart, istart):
      g_wait(j, b)
      s_start(j, b)
      if swait:
        s_wait(j - GDEPTH, (b + GDEPTH) % NBUF)
      if gstart:
        i_wait(j + GDEPTH, (b + GDEPTH) % NBUF)
        g_start(j + GDEPTH, (b + GDEPTH) % NBUF)
      if istart:
        i_start(j + GDEPTH + 1, (b + GDEPTH + 1) % NBUF)

    for j in range(GDEPTH + 1):
      i_start(j, j % NBUF)
    for j in range(GDEPTH):
      i_wait(j, j % NBUF)
      g_start(j, j % NBUF)

    zero16 = jnp.zeros((16,), jnp.float32)

    def zfill_body(r, _):
      for q in range(W // 16):
        rows[2][r, pl.ds(q * 16, 16)] = zero16
        rows[3][r, pl.ds(q * 16, 16)] = zero16
      return _
    lax.fori_loop(0, CHUNK, zfill_body, None)
    for i in range(ROWS_PT // CHUNK):
      b = 2 + i % 2
      pltpu.async_copy(rows[b], agg.at[pl.ds(rbase + i * CHUNK, CHUNK)],
                       ssem[b])
    for i in range(ROWS_PT // CHUNK):
      b = 2 + i % 2
      pltpu.make_async_copy(rows[b], agg.at[pl.ds(rbase + i * CHUNK, CHUNK)],
                            ssem[b]).wait()
    plsc.subcore_barrier()

    for j in range(NBUF):
      slot(j, j % NBUF, swait=(j >= GDEPTH), gstart=True, istart=True)

    def ring_body(g, _):
      j0 = g * NBUF
      for b in range(NBUF):
        slot(j0 + b, b, swait=True, gstart=True, istart=True)
      return _
    lax.fori_loop(1, (NCHUNK - NBUF - 1) // NBUF, ring_body, None)

    for j in range(((NCHUNK - NBUF - 1) // NBUF) * NBUF, NCHUNK):
      slot(j, j % NBUF, swait=True,
           gstart=(j + GDEPTH < NCHUNK),
           istart=(j + GDEPTH + 1 < NCHUNK))
    for j in range(NCHUNK - GDEPTH, NCHUNK):
      s_wait(j, j % NBUF)
    plsc.subcore_barrier()

    obase = cid * NP + rbase
    pltpu.sync_copy(agg.at[pl.ds(rbase, ROWS_PT)],
                    out.at[pl.ds(obase, ROWS_PT)])

  return scatter_kernel


_scatter128 = _make_edge_scatter(D)

_BR = 1024


def _mm_body(x_ref, w_ref, o_ref):
  o_ref[...] = jnp.dot(x_ref[...], w_ref[...],
                       preferred_element_type=jnp.float32)


def _combine_mm_body(p0_ref, p1_ref, b_ref, w_ref, o_ref):
  h = jnp.maximum(p0_ref[...] + p1_ref[...] + b_ref[...], 0.0)
  o_ref[...] = jnp.dot(h, w_ref[...], preferred_element_type=jnp.float32)


def _combine_body(p0_ref, p1_ref, b_ref, o_ref):
  o_ref[...] = jnp.maximum(p0_ref[...] + p1_ref[...] + b_ref[...], 0.0)


def _final_body(p0_ref, p1_ref, w_ref, b_ref, o_ref):
  agg = p0_ref[...] + p1_ref[...]
  v = jnp.dot(agg, w_ref[...],
              preferred_element_type=jnp.float32) + b_ref[...]
  col = lax.broadcasted_iota(jnp.int32, v.shape, 1)
  valid = col < N_CLASS
  vm = jnp.where(valid, v, -jnp.inf)
  m = jnp.max(vm, axis=1, keepdims=True)
  ex = jnp.where(valid, jnp.exp(v - m), 0.0)
  lse = jnp.log(jnp.sum(ex, axis=1, keepdims=True)) + m
  o_ref[...] = v - lse


def _tc_matmul(x, w):
  n, k = x.shape
  kw, m = w.shape
  return pl.pallas_call(
      _mm_body,
      grid=(n // _BR,),
      in_specs=[pl.BlockSpec((_BR, k), lambda i: (i, 0)),
                pl.BlockSpec((kw, m), lambda i: (0, 0))],
      out_specs=pl.BlockSpec((_BR, m), lambda i: (i, 0)),
      out_shape=jax.ShapeDtypeStruct((n, m), jnp.float32),
  )(x, w)


def _tc_combine_mm(p0, p1, b, w):
  n, k = p0.shape
  kw, m = w.shape
  return pl.pallas_call(
      _combine_mm_body,
      grid=(n // _BR,),
      in_specs=[pl.BlockSpec((_BR, k), lambda i: (i, 0)),
                pl.BlockSpec((_BR, k), lambda i: (i, 0)),
                pl.BlockSpec((1, k), lambda i: (0, 0)),
                pl.BlockSpec((kw, m), lambda i: (0, 0))],
      out_specs=pl.BlockSpec((_BR, m), lambda i: (i, 0)),
      out_shape=jax.ShapeDtypeStruct((n, m), jnp.float32),
  )(p0, p1, b, w)


def _tc_combine(p0, p1, b):
  n, k = p0.shape
  return pl.pallas_call(
      _combine_body,
      grid=(n // _BR,),
      in_specs=[pl.BlockSpec((_BR, k), lambda i: (i, 0)),
                pl.BlockSpec((_BR, k), lambda i: (i, 0)),
                pl.BlockSpec((1, k), lambda i: (0, 0))],
      out_specs=pl.BlockSpec((_BR, k), lambda i: (i, 0)),
      out_shape=jax.ShapeDtypeStruct((n, k), jnp.float32),
  )(p0, p1, b)


def _tc_final(p0, p1, w, b):
  n, k = p0.shape
  kw, m = w.shape
  return pl.pallas_call(
      _final_body,
      grid=(n // _BR,),
      in_specs=[pl.BlockSpec((_BR, k), lambda i: (i, 0)),
                pl.BlockSpec((_BR, k), lambda i: (i, 0)),
                pl.BlockSpec((kw, m), lambda i: (0, 0)),
                pl.BlockSpec((1, m), lambda i: (0, 0))],
      out_specs=pl.BlockSpec((_BR, m), lambda i: (i, 0)),
      out_shape=jax.ShapeDtypeStruct((n, m), jnp.float32),
  )(p0, p1, w, b)


def kernel(x, edge_index, W0, b0, W1, b1, Wout, bout):
  src = edge_index[0].astype(jnp.int32)
  dst = edge_index[1].astype(jnp.int32)
  xp = jnp.pad(x, ((0, NP - N_NODES), (0, 0)))
  wout_p = jnp.pad(Wout, ((0, 0), (0, WOUT_PAD - N_CLASS)))
  bout_p = jnp.pad(bout, (0, WOUT_PAD - N_CLASS)).reshape(1, WOUT_PAD)
  b0r = b0.reshape(1, D)
  b1r = b1.reshape(1, D)

  support0 = _tc_matmul(xp, W0)
  p0 = _scatter128(support0, src, dst)
  support1 = _tc_combine_mm(p0[:NP], p0[NP:], b0r, W1)
  p1 = _scatter128(support1, src, dst)
  h2 = _tc_combine(p1[:NP], p1[NP:], b1r)
  p2 = _scatter128(h2, src, dst)
  pred = _tc_final(p2[:NP], p2[NP:], wout_p, bout_p)
  return pred[:N_NODES, :N_CLASS]
